# Initial kernel scaffold; baseline (speedup 1.0000x reference)
import jax
import jax.numpy as jnp
from jax.experimental import pallas as pl


def kernel(feat, edge_index, y, mask, val_idx, test_idx, W):
    raise NotImplementedError("write your pallas kernel here")


# speedup vs baseline: 133.0838x; 133.0838x over previous
import functools

import jax
import jax.numpy as jnp
from jax import lax
from jax.experimental import pallas as pl
from jax.experimental.pallas import tpu as pltpu
from jax.experimental.pallas import tpu_sc as plsc

NC = 2
NS = 16
NT = NC * NS
CP = 48


def _sc_feat_body(NPAD, NCH, KC,
                  feat_h, src_h, dst_h, zf_h,
                  o_pf, src_v, dst_v, rows_f, sem0, acc_f):
    c = lax.axis_index("c")
    s = lax.axis_index("s")
    wid = s * NC + c
    R = NPAD // NS
    st = pl.ds(s * R, R)

    pltpu.sync_copy(zf_h.at[st], acc_f.at[st])
    pltpu.sync_copy(src_h.at[wid], src_v)
    pltpu.sync_copy(dst_h.at[wid], dst_v)
    plsc.subcore_barrier()

    def step(j, carry):
        pltpu.async_copy(feat_h.at[src_v.at[j]], rows_f, sem0).wait()
        pltpu.sync_copy(rows_f, acc_f.at[dst_v.at[j]], add=True)
        return carry

    lax.fori_loop(0, NCH, step, 0)
    plsc.subcore_barrier()
    pltpu.sync_copy(acc_f.at[st], o_pf.at[c, st])


def _sc_label_body(NPAD, NCH, KC, LCH, UCH, UW,
                   src_h, dst_h, oh_h, lrow_h, unl_h, zl_h, zd_h, ones_h,
                   o_pl, o_pd, o_lab1, o_wr,
                   src_v, dst_v, rows_l, ones_v, lab_build,
                   lrow_v, unl_v, sem0, acc_l, acc_d):
    c = lax.axis_index("c")
    s = lax.axis_index("s")
    wid = s * NC + c
    R = NPAD // NS
    st = pl.ds(s * R, R)

    pltpu.sync_copy(zl_h.at[st], acc_l.at[st])
    pltpu.sync_copy(zd_h.at[st], acc_d.at[st])
    pltpu.sync_copy(lrow_h.at[s], lrow_v)
    pltpu.sync_copy(unl_h.at[s], unl_v)
    pltpu.sync_copy(ones_h, ones_v)
    plsc.subcore_barrier()

    for ch in range(LCH):
        pltpu.async_copy(oh_h.at[lrow_v.at[ch]], lab_build, sem0).wait()
        pltpu.sync_copy(lab_build, acc_l.at[lrow_v.at[ch]])

    for ch in range(UCH):
        pltpu.sync_copy(ones_v.at[pl.ds(0, UW)], acc_d.at[unl_v.at[ch]])
    plsc.subcore_barrier()

    pltpu.sync_copy(acc_l.at[st], o_lab1.at[st])
    pltpu.sync_copy(acc_d.at[st], o_wr.at[st])
    pltpu.sync_copy(zl_h.at[st], acc_l.at[st])
    pltpu.sync_copy(zd_h.at[st], acc_d.at[st])
    plsc.subcore_barrier()

    pltpu.sync_copy(src_h.at[wid], src_v)
    pltpu.sync_copy(dst_h.at[wid], dst_v)

    def step(j, carry):
        pltpu.async_copy(o_lab1.at[src_v.at[j]], rows_l, sem0).wait()
        pltpu.sync_copy(rows_l, acc_l.at[dst_v.at[j]], add=True)
        pltpu.sync_copy(ones_v.at[pl.ds(0, KC)], acc_d.at[dst_v.at[j]],
                        add=True)
        return carry

    lax.fori_loop(0, NCH, step, 0)
    plsc.subcore_barrier()

    pltpu.sync_copy(acc_l.at[st], o_pl.at[c, st])
    pltpu.sync_copy(acc_d.at[st], o_pd.at[c, st])


def _sc_label2_body(NPAD, NCH, KC,
                    lab2_h, src_h, dst_h, zl_h,
                    o_pl, src_v, dst_v, rows_l, sem0, acc_l):
    c = lax.axis_index("c")
    s = lax.axis_index("s")
    wid = s * NC + c
    R = NPAD // NS
    st = pl.ds(s * R, R)

    pltpu.sync_copy(zl_h.at[st], acc_l.at[st])
    pltpu.sync_copy(src_h.at[wid], src_v)
    pltpu.sync_copy(dst_h.at[wid], dst_v)
    plsc.subcore_barrier()

    def step(j, carry):
        pltpu.async_copy(lab2_h.at[src_v.at[j]], rows_l, sem0).wait()
        pltpu.sync_copy(rows_l, acc_l.at[dst_v.at[j]], add=True)
        return carry

    lax.fori_loop(0, NCH, step, 0)
    plsc.subcore_barrier()
    pltpu.sync_copy(acc_l.at[st], o_pl.at[c, st])


def _tc_mid_body(pf, plr, pd, lab1, wr, w1, w2, o_pa, o_l2):
    af = pf[0] + pf[1]
    al = plr[0] + plr[1]
    dg = pd[0] + pd[1]
    inv = 1.0 / jnp.maximum(dg[:, 0:1], 1.0)
    pa = jnp.dot(af * inv, w1[...])
    pred1 = pa + jnp.dot(al[:, :40] * inv, w2[...])
    m = jnp.max(pred1, axis=-1, keepdims=True)
    e = jnp.exp(pred1 - m)
    sm = e / jnp.sum(e, axis=-1, keepdims=True)
    l2 = jnp.where(wr[:, 0:1] > 0.0, sm, lab1[:, :40])
    o_pa[...] = pa
    o_l2[...] = jnp.pad(l2, ((0, 0), (0, CP - 40)))


def _tc_final_body(pl2, pd, pa, w2, o):
    al = pl2[0] + pl2[1]
    dg = pd[0] + pd[1]
    inv = 1.0 / jnp.maximum(dg[:, 0:1], 1.0)
    o[...] = pa[...] + jnp.dot(al[:, :40] * inv, w2[...])


def kernel(feat, edge_index, y, mask, val_idx, test_idx, W):
    N, F = feat.shape
    C = W.shape[1]
    E = edge_index.shape[1]
    NPAD = -(-(N + 1) // (8 * NS)) * (8 * NS)
    EPT = E // NT
    KC = 125
    NCH = EPT // KC
    assert EPT % KC == 0 and E % NT == 0

    split = jax.random.uniform(jax.random.key(42), mask.shape) < 0.5
    lrows = jnp.where(split, mask, N).astype(jnp.int32)
    LM = mask.shape[0]
    LPAD = -(-LM // (NS * 96)) * (NS * 96)
    LCH = LPAD // NS // 96
    lrows_p = jnp.full((LPAD,), N, jnp.int32).at[:LM].set(lrows)
    lrows3 = lrows_p.reshape(NS, LCH, 96)

    unl_idx = jnp.concatenate([mask, val_idx, test_idx])
    unl_valid = jnp.concatenate([
        ~split,
        jnp.ones(val_idx.shape, bool),
        jnp.ones(test_idx.shape, bool),
    ])
    unl_rows = jnp.where(unl_valid, unl_idx, N).astype(jnp.int32)
    UM = unl_rows.shape[0]
    UW = 80
    UPAD = -(-UM // (NS * UW)) * (NS * UW)
    UCH = UPAD // NS // UW
    unl_p = jnp.full((UPAD,), N, jnp.int32).at[:UM].set(unl_rows)
    unl3 = unl_p.reshape(NS, UCH, UW)

    src3 = edge_index[0].reshape(NT, NCH, KC).astype(jnp.int32)
    dst3 = edge_index[1].reshape(NT, NCH, KC).astype(jnp.int32)
    y_p = jnp.full((NPAD,), -1, jnp.int32).at[:N].set(y)
    oh_h = jnp.pad((y_p[:, None] == jnp.arange(C)[None, :]).astype(jnp.float32),
                   ((0, 0), (0, CP - C)))
    zf_h = jnp.zeros((NPAD, F), jnp.float32)
    zl_h = jnp.zeros((NPAD, CP), jnp.float32)
    zd_h = jnp.zeros((NPAD, 16), jnp.float32)
    ones_h = jnp.ones((128, 16), jnp.float32)
    w1 = W[:F]
    w2 = W[F:]

    f32 = jnp.float32
    mesh = plsc.VectorSubcoreMesh(core_axis_name="c", subcore_axis_name="s")
    sc_params = pltpu.CompilerParams(use_tc_tiling_on_sc=False)

    sc_feat = functools.partial(
        pl.kernel,
        functools.partial(_sc_feat_body, NPAD, NCH, KC),
        out_type=[jax.ShapeDtypeStruct((NC, NPAD, F), f32)],
        mesh=mesh,
        compiler_params=sc_params,
        scratch_types=[
            pltpu.VMEM((NCH, KC), jnp.int32),
            pltpu.VMEM((NCH, KC), jnp.int32),
            pltpu.VMEM((KC, F), f32),
            pltpu.SemaphoreType.DMA,
            pltpu.VMEM_SHARED((NPAD, F), f32),
        ],
    )()
    (pf,) = sc_feat(feat, src3, dst3, zf_h)

    sc_label = functools.partial(
        pl.kernel,
        functools.partial(_sc_label_body, NPAD, NCH, KC, LCH, UCH, UW),
        out_type=[
            jax.ShapeDtypeStruct((NC, NPAD, CP), f32),
            jax.ShapeDtypeStruct((NC, NPAD, 16), f32),
            jax.ShapeDtypeStruct((NPAD, CP), f32),
            jax.ShapeDtypeStruct((NPAD, 16), f32),
        ],
        mesh=mesh,
        compiler_params=sc_params,
        scratch_types=[
            pltpu.VMEM((NCH, KC), jnp.int32),
            pltpu.VMEM((NCH, KC), jnp.int32),
            pltpu.VMEM((KC, CP), f32),
            pltpu.VMEM((128, 16), f32),
            pltpu.VMEM((96, CP), f32),
            pltpu.VMEM((LCH, 96), jnp.int32),
            pltpu.VMEM((UCH, UW), jnp.int32),
            pltpu.SemaphoreType.DMA,
            pltpu.VMEM_SHARED((NPAD, CP), f32),
            pltpu.VMEM_SHARED((NPAD, 16), f32),
        ],
    )()
    plr, pd, lab1, wr = sc_label(
        src3, dst3, oh_h, lrows3, unl3, zl_h, zd_h, ones_h)

    BR = NPAD // 8
    pa, lab2 = pl.pallas_call(
        _tc_mid_body,
        grid=(NPAD // BR,),
        in_specs=[
            pl.BlockSpec((NC, BR, F), lambda i: (0, i, 0)),
            pl.BlockSpec((NC, BR, CP), lambda i: (0, i, 0)),
            pl.BlockSpec((NC, BR, 16), lambda i: (0, i, 0)),
            pl.BlockSpec((BR, CP), lambda i: (i, 0)),
            pl.BlockSpec((BR, 16), lambda i: (i, 0)),
            pl.BlockSpec((F, C), lambda i: (0, 0)),
            pl.BlockSpec((C, C), lambda i: (0, 0)),
        ],
        out_specs=[
            pl.BlockSpec((BR, C), lambda i: (i, 0)),
            pl.BlockSpec((BR, CP), lambda i: (i, 0)),
        ],
        out_shape=[
            jax.ShapeDtypeStruct((NPAD, C), f32),
            jax.ShapeDtypeStruct((NPAD, CP), f32),
        ],
    )(pf, plr, pd, lab1, wr, w1, w2)

    sc_lab2 = functools.partial(
        pl.kernel,
        functools.partial(_sc_label2_body, NPAD, NCH, KC),
        out_type=[jax.ShapeDtypeStruct((NC, NPAD, CP), f32)],
        mesh=mesh,
        compiler_params=sc_params,
        scratch_types=[
            pltpu.VMEM((NCH, KC), jnp.int32),
            pltpu.VMEM((NCH, KC), jnp.int32),
            pltpu.VMEM((KC, CP), f32),
            pltpu.SemaphoreType.DMA,
            pltpu.VMEM_SHARED((NPAD, CP), f32),
        ],
    )()
    (pl2,) = sc_lab2(lab2, src3, dst3, zl_h)

    pred2 = pl.pallas_call(
        _tc_final_body,
        grid=(NPAD // BR,),
        in_specs=[
            pl.BlockSpec((NC, BR, CP), lambda i: (0, i, 0)),
            pl.BlockSpec((NC, BR, 16), lambda i: (0, i, 0)),
            pl.BlockSpec((BR, C), lambda i: (i, 0)),
            pl.BlockSpec((C, C), lambda i: (0, 0)),
        ],
        out_specs=pl.BlockSpec((BR, C), lambda i: (i, 0)),
        out_shape=jax.ShapeDtypeStruct((NPAD, C), f32),
    )(pl2, pd, pa, w2)

    return pred2[:N]


# speedup vs baseline: 174.8179x; 1.3136x over previous
import functools

import jax
import jax.numpy as jnp
from jax import lax
from jax.experimental import pallas as pl
from jax.experimental.pallas import tpu as pltpu
from jax.experimental.pallas import tpu_sc as plsc

NC = 2
NS = 16
NT = NC * NS
CP = 48


def _edge_pipeline(NCH, tab_h, src_v, b0, b1, s0, s1, scat0, scat1):
    pltpu.async_copy(tab_h.at[src_v.at[0]], b0, s0)

    def step(jj, carry):
        j0 = jj * 2
        j1 = j0 + 1
        pltpu.async_copy(tab_h.at[src_v.at[j1]], b1, s1)
        pltpu.make_async_copy(tab_h.at[src_v.at[j0]], b0, s0).wait()
        scat0(j0)

        @pl.when(j0 + 2 < NCH)
        def _():
            pltpu.async_copy(tab_h.at[src_v.at[j0 + 2]], b0, s0)

        pltpu.make_async_copy(tab_h.at[src_v.at[j1]], b1, s1).wait()
        scat1(j1)
        return carry

    lax.fori_loop(0, NCH // 2, step, 0)


def _sc_feat_body(NPAD, NCH, KC,
                  feat_h, src_h, dst_h, zf_h,
                  o_pf, src_v, dst_v, rows_f0, rows_f1, sem0, sem1, acc_f):
    c = lax.axis_index("c")
    s = lax.axis_index("s")
    wid = s * NC + c
    R = NPAD // NS
    st = pl.ds(s * R, R)

    pltpu.sync_copy(zf_h.at[st], acc_f.at[st])
    pltpu.sync_copy(src_h.at[wid], src_v)
    pltpu.sync_copy(dst_h.at[wid], dst_v)
    plsc.subcore_barrier()

    def scat(buf):
        def f(j):
            pltpu.sync_copy(buf, acc_f.at[dst_v.at[j]], add=True)
        return f

    _edge_pipeline(NCH, feat_h, src_v, rows_f0, rows_f1, sem0, sem1,
                   scat(rows_f0), scat(rows_f1))
    plsc.subcore_barrier()
    pltpu.sync_copy(acc_f.at[st], o_pf.at[c, st])


def _sc_label_body(NPAD, NCH, KC, LCH, UCH, UW,
                   src_h, dst_h, oh_h, lrow_h, unl_h, zl_h, zd_h, ones_h,
                   o_pl, o_pd, o_lab1, o_wr,
                   src_v, dst_v, rows_l0, rows_l1, ones_v, lab_build,
                   lrow_v, unl_v, sem0, sem1, acc_l, acc_d):
    c = lax.axis_index("c")
    s = lax.axis_index("s")
    wid = s * NC + c
    R = NPAD // NS
    st = pl.ds(s * R, R)

    pltpu.sync_copy(zl_h.at[st], acc_l.at[st])
    pltpu.sync_copy(zd_h.at[st], acc_d.at[st])
    pltpu.sync_copy(lrow_h.at[s], lrow_v)
    pltpu.sync_copy(unl_h.at[s], unl_v)
    pltpu.sync_copy(ones_h, ones_v)
    plsc.subcore_barrier()

    for ch in range(LCH):
        pltpu.async_copy(oh_h.at[lrow_v.at[ch]], lab_build, sem0).wait()
        pltpu.sync_copy(lab_build, acc_l.at[lrow_v.at[ch]])

    for ch in range(UCH):
        pltpu.sync_copy(ones_v.at[pl.ds(0, UW)], acc_d.at[unl_v.at[ch]])
    plsc.subcore_barrier()

    pltpu.sync_copy(acc_l.at[st], o_lab1.at[st])
    pltpu.sync_copy(acc_d.at[st], o_wr.at[st])
    pltpu.sync_copy(zl_h.at[st], acc_l.at[st])
    pltpu.sync_copy(zd_h.at[st], acc_d.at[st])
    plsc.subcore_barrier()

    pltpu.sync_copy(src_h.at[wid], src_v)
    pltpu.sync_copy(dst_h.at[wid], dst_v)

    def scat(buf):
        def f(j):
            pltpu.sync_copy(buf, acc_l.at[dst_v.at[j]], add=True)
            pltpu.sync_copy(ones_v.at[pl.ds(0, KC)], acc_d.at[dst_v.at[j]],
                            add=True)
        return f

    _edge_pipeline(NCH, o_lab1, src_v, rows_l0, rows_l1, sem0, sem1,
                   scat(rows_l0), scat(rows_l1))
    plsc.subcore_barrier()

    pltpu.sync_copy(acc_l.at[st], o_pl.at[c, st])
    pltpu.sync_copy(acc_d.at[st], o_pd.at[c, st])


def _sc_label2_body(NPAD, NCH, KC,
                    lab2_h, src_h, dst_h, zl_h,
                    o_pl, src_v, dst_v, rows_l0, rows_l1, sem0, sem1, acc_l):
    c = lax.axis_index("c")
    s = lax.axis_index("s")
    wid = s * NC + c
    R = NPAD // NS
    st = pl.ds(s * R, R)

    pltpu.sync_copy(zl_h.at[st], acc_l.at[st])
    pltpu.sync_copy(src_h.at[wid], src_v)
    pltpu.sync_copy(dst_h.at[wid], dst_v)
    plsc.subcore_barrier()

    def scat(buf):
        def f(j):
            pltpu.sync_copy(buf, acc_l.at[dst_v.at[j]], add=True)
        return f

    _edge_pipeline(NCH, lab2_h, src_v, rows_l0, rows_l1, sem0, sem1,
                   scat(rows_l0), scat(rows_l1))
    plsc.subcore_barrier()
    pltpu.sync_copy(acc_l.at[st], o_pl.at[c, st])


def _tc_mid_body(pf, plr, pd, lab1, wr, w1, w2, o_pa, o_l2):
    af = pf[0] + pf[1]
    al = plr[0] + plr[1]
    dg = pd[0] + pd[1]
    inv = 1.0 / jnp.maximum(dg[:, 0:1], 1.0)
    pa = jnp.dot(af * inv, w1[...])
    pred1 = pa + jnp.dot(al[:, :40] * inv, w2[...])
    m = jnp.max(pred1, axis=-1, keepdims=True)
    e = jnp.exp(pred1 - m)
    sm = e / jnp.sum(e, axis=-1, keepdims=True)
    l2 = jnp.where(wr[:, 0:1] > 0.0, sm, lab1[:, :40])
    o_pa[...] = pa
    o_l2[...] = jnp.pad(l2, ((0, 0), (0, CP - 40)))


def _tc_final_body(pl2, pd, pa, w2, o):
    al = pl2[0] + pl2[1]
    dg = pd[0] + pd[1]
    inv = 1.0 / jnp.maximum(dg[:, 0:1], 1.0)
    o[...] = pa[...] + jnp.dot(al[:, :40] * inv, w2[...])


def kernel(feat, edge_index, y, mask, val_idx, test_idx, W):
    N, F = feat.shape
    C = W.shape[1]
    E = edge_index.shape[1]
    NPAD = -(-(N + 1) // (8 * NS)) * (8 * NS)
    EPT = E // NT
    KC = 100
    NCH = EPT // KC
    assert EPT % KC == 0 and E % NT == 0

    split = jax.random.uniform(jax.random.key(42), mask.shape) < 0.5
    lrows = jnp.where(split, mask, N).astype(jnp.int32)
    LM = mask.shape[0]
    LPAD = -(-LM // (NS * 96)) * (NS * 96)
    LCH = LPAD // NS // 96
    lrows_p = jnp.full((LPAD,), N, jnp.int32).at[:LM].set(lrows)
    lrows3 = lrows_p.reshape(NS, LCH, 96)

    unl_idx = jnp.concatenate([mask, val_idx, test_idx])
    unl_valid = jnp.concatenate([
        ~split,
        jnp.ones(val_idx.shape, bool),
        jnp.ones(test_idx.shape, bool),
    ])
    unl_rows = jnp.where(unl_valid, unl_idx, N).astype(jnp.int32)
    UM = unl_rows.shape[0]
    UW = 80
    UPAD = -(-UM // (NS * UW)) * (NS * UW)
    UCH = UPAD // NS // UW
    unl_p = jnp.full((UPAD,), N, jnp.int32).at[:UM].set(unl_rows)
    unl3 = unl_p.reshape(NS, UCH, UW)

    src3 = edge_index[0].reshape(NT, NCH, KC).astype(jnp.int32)
    dst3 = edge_index[1].reshape(NT, NCH, KC).astype(jnp.int32)
    y_p = jnp.full((NPAD,), -1, jnp.int32).at[:N].set(y)
    oh_h = jnp.pad((y_p[:, None] == jnp.arange(C)[None, :]).astype(jnp.float32),
                   ((0, 0), (0, CP - C)))
    zf_h = jnp.zeros((NPAD, F), jnp.float32)
    zl_h = jnp.zeros((NPAD, CP), jnp.float32)
    zd_h = jnp.zeros((NPAD, 16), jnp.float32)
    ones_h = jnp.ones((128, 16), jnp.float32)
    w1 = W[:F]
    w2 = W[F:]

    f32 = jnp.float32
    mesh = plsc.VectorSubcoreMesh(core_axis_name="c", subcore_axis_name="s")
    sc_params = pltpu.CompilerParams(use_tc_tiling_on_sc=False)

    sc_feat = functools.partial(
        pl.kernel,
        functools.partial(_sc_feat_body, NPAD, NCH, KC),
        out_type=[jax.ShapeDtypeStruct((NC, NPAD, F), f32)],
        mesh=mesh,
        compiler_params=sc_params,
        scratch_types=[
            pltpu.VMEM((NCH, KC), jnp.int32),
            pltpu.VMEM((NCH, KC), jnp.int32),
            pltpu.VMEM((KC, F), f32),
            pltpu.VMEM((KC, F), f32),
            pltpu.SemaphoreType.DMA,
            pltpu.SemaphoreType.DMA,
            pltpu.VMEM_SHARED((NPAD, F), f32),
        ],
    )()
    (pf,) = sc_feat(feat, src3, dst3, zf_h)

    sc_label = functools.partial(
        pl.kernel,
        functools.partial(_sc_label_body, NPAD, NCH, KC, LCH, UCH, UW),
        out_type=[
            jax.ShapeDtypeStruct((NC, NPAD, CP), f32),
            jax.ShapeDtypeStruct((NC, NPAD, 16), f32),
            jax.ShapeDtypeStruct((NPAD, CP), f32),
            jax.ShapeDtypeStruct((NPAD, 16), f32),
        ],
        mesh=mesh,
        compiler_params=sc_params,
        scratch_types=[
            pltpu.VMEM((NCH, KC), jnp.int32),
            pltpu.VMEM((NCH, KC), jnp.int32),
            pltpu.VMEM((KC, CP), f32),
            pltpu.VMEM((KC, CP), f32),
            pltpu.VMEM((128, 16), f32),
            pltpu.VMEM((96, CP), f32),
            pltpu.VMEM((LCH, 96), jnp.int32),
            pltpu.VMEM((UCH, UW), jnp.int32),
            pltpu.SemaphoreType.DMA,
            pltpu.SemaphoreType.DMA,
            pltpu.VMEM_SHARED((NPAD, CP), f32),
            pltpu.VMEM_SHARED((NPAD, 16), f32),
        ],
    )()
    plr, pd, lab1, wr = sc_label(
        src3, dst3, oh_h, lrows3, unl3, zl_h, zd_h, ones_h)

    BR = NPAD // 8
    pa, lab2 = pl.pallas_call(
        _tc_mid_body,
        grid=(NPAD // BR,),
        in_specs=[
            pl.BlockSpec((NC, BR, F), lambda i: (0, i, 0)),
            pl.BlockSpec((NC, BR, CP), lambda i: (0, i, 0)),
            pl.BlockSpec((NC, BR, 16), lambda i: (0, i, 0)),
            pl.BlockSpec((BR, CP), lambda i: (i, 0)),
            pl.BlockSpec((BR, 16), lambda i: (i, 0)),
            pl.BlockSpec((F, C), lambda i: (0, 0)),
            pl.BlockSpec((C, C), lambda i: (0, 0)),
        ],
        out_specs=[
            pl.BlockSpec((BR, C), lambda i: (i, 0)),
            pl.BlockSpec((BR, CP), lambda i: (i, 0)),
        ],
        out_shape=[
            jax.ShapeDtypeStruct((NPAD, C), f32),
            jax.ShapeDtypeStruct((NPAD, CP), f32),
        ],
    )(pf, plr, pd, lab1, wr, w1, w2)

    sc_lab2 = functools.partial(
        pl.kernel,
        functools.partial(_sc_label2_body, NPAD, NCH, KC),
        out_type=[jax.ShapeDtypeStruct((NC, NPAD, CP), f32)],
        mesh=mesh,
        compiler_params=sc_params,
        scratch_types=[
            pltpu.VMEM((NCH, KC), jnp.int32),
            pltpu.VMEM((NCH, KC), jnp.int32),
            pltpu.VMEM((KC, CP), f32),
            pltpu.VMEM((KC, CP), f32),
            pltpu.SemaphoreType.DMA,
            pltpu.SemaphoreType.DMA,
            pltpu.VMEM_SHARED((NPAD, CP), f32),
        ],
    )()
    (pl2,) = sc_lab2(lab2, src3, dst3, zl_h)

    pred2 = pl.pallas_call(
        _tc_final_body,
        grid=(NPAD // BR,),
        in_specs=[
            pl.BlockSpec((NC, BR, CP), lambda i: (0, i, 0)),
            pl.BlockSpec((NC, BR, 16), lambda i: (0, i, 0)),
            pl.BlockSpec((BR, C), lambda i: (i, 0)),
            pl.BlockSpec((C, C), lambda i: (0, 0)),
        ],
        out_specs=pl.BlockSpec((BR, C), lambda i: (i, 0)),
        out_shape=jax.ShapeDtypeStruct((NPAD, C), f32),
    )(pl2, pd, pa, w2)

    return pred2[:N]


# speedup vs baseline: 176.8052x; 1.0114x over previous
import functools

import jax
import jax.numpy as jnp
from jax import lax
from jax.experimental import pallas as pl
from jax.experimental.pallas import tpu as pltpu
from jax.experimental.pallas import tpu_sc as plsc

NC = 2
NS = 16
NT = NC * NS
CP = 48


def _edge_pipeline(NCH, tab_h, src_v, b0, b1, s0, s1, scat0, scat1):
    pltpu.async_copy(tab_h.at[src_v.at[0]], b0, s0)

    def step(jj, carry):
        j0 = jj * 2
        j1 = j0 + 1
        pltpu.async_copy(tab_h.at[src_v.at[j1]], b1, s1)
        pltpu.make_async_copy(tab_h.at[src_v.at[j0]], b0, s0).wait()
        scat0(j0)

        @pl.when(j0 + 2 < NCH)
        def _():
            pltpu.async_copy(tab_h.at[src_v.at[j0 + 2]], b0, s0)

        pltpu.make_async_copy(tab_h.at[src_v.at[j1]], b1, s1).wait()
        scat1(j1)
        return carry

    lax.fori_loop(0, NCH // 2, step, 0)


def _sc_feat_body(NPAD, NCH, KC,
                  feat_h, src_h, dst_h, zf_h,
                  o_pf, src_v, dst_v, rows_f0, rows_f1, sem0, sem1, acc_f):
    c = lax.axis_index("c")
    s = lax.axis_index("s")
    wid = s * NC + c
    R = NPAD // NS
    st = pl.ds(s * R, R)

    pltpu.sync_copy(zf_h.at[st], acc_f.at[st])
    pltpu.sync_copy(src_h.at[wid], src_v)
    pltpu.sync_copy(dst_h.at[wid], dst_v)
    plsc.subcore_barrier()

    def scat(buf):
        def f(j):
            pltpu.sync_copy(buf, acc_f.at[dst_v.at[j]], add=True)
        return f

    _edge_pipeline(NCH, feat_h, src_v, rows_f0, rows_f1, sem0, sem1,
                   scat(rows_f0), scat(rows_f1))
    plsc.subcore_barrier()
    pltpu.sync_copy(acc_f.at[st], o_pf.at[c, st])


def _sc_label_body(NPAD, NCH, KC, LCH, UCH, UW,
                   src_h, dst_h, oh_h, lrow_h, unl_h, zl_h, zc_h, zd_h,
                   ones_h,
                   o_pl, o_lab1, o_wr,
                   src_v, dst_v, rows_l0, rows_l1, ones_v, lab_build,
                   lrow_v, unl_v, sem0, sem1, acc_l, acc_d):
    c = lax.axis_index("c")
    s = lax.axis_index("s")
    wid = s * NC + c
    R = NPAD // NS
    st = pl.ds(s * R, R)

    pltpu.sync_copy(zc_h.at[st], acc_l.at[st])
    pltpu.sync_copy(zd_h.at[st], acc_d.at[st])
    pltpu.sync_copy(lrow_h.at[s], lrow_v)
    pltpu.sync_copy(unl_h.at[s], unl_v)
    pltpu.sync_copy(ones_h, ones_v)
    plsc.subcore_barrier()

    for ch in range(LCH):
        pltpu.async_copy(oh_h.at[lrow_v.at[ch]], lab_build, sem0).wait()
        pltpu.sync_copy(lab_build, acc_l.at[lrow_v.at[ch]])

    for ch in range(UCH):
        pltpu.sync_copy(ones_v.at[pl.ds(0, UW)], acc_d.at[unl_v.at[ch]])
    plsc.subcore_barrier()

    pltpu.sync_copy(acc_l.at[st], o_lab1.at[st])
    pltpu.sync_copy(acc_d.at[st], o_wr.at[st])
    pltpu.sync_copy(zl_h.at[st], acc_l.at[st])
    plsc.subcore_barrier()

    pltpu.sync_copy(src_h.at[wid], src_v)
    pltpu.sync_copy(dst_h.at[wid], dst_v)

    def scat(buf):
        def f(j):
            pltpu.sync_copy(buf, acc_l.at[dst_v.at[j]], add=True)
        return f

    _edge_pipeline(NCH, o_lab1, src_v, rows_l0, rows_l1, sem0, sem1,
                   scat(rows_l0), scat(rows_l1))
    plsc.subcore_barrier()

    pltpu.sync_copy(acc_l.at[st], o_pl.at[c, st])


def _sc_label2_body(NPAD, NCH, KC,
                    lab2_h, src_h, dst_h, zl_h,
                    o_pl, src_v, dst_v, rows_l0, rows_l1, sem0, sem1, acc_l):
    c = lax.axis_index("c")
    s = lax.axis_index("s")
    wid = s * NC + c
    R = NPAD // NS
    st = pl.ds(s * R, R)

    pltpu.sync_copy(zl_h.at[st], acc_l.at[st])
    pltpu.sync_copy(src_h.at[wid], src_v)
    pltpu.sync_copy(dst_h.at[wid], dst_v)
    plsc.subcore_barrier()

    def scat(buf):
        def f(j):
            pltpu.sync_copy(buf, acc_l.at[dst_v.at[j]], add=True)
        return f

    _edge_pipeline(NCH, lab2_h, src_v, rows_l0, rows_l1, sem0, sem1,
                   scat(rows_l0), scat(rows_l1))
    plsc.subcore_barrier()
    pltpu.sync_copy(acc_l.at[st], o_pl.at[c, st])


def _tc_mid_body(pf, plr, lab1, wr, w1, w2, o_pa, o_l2):
    af = pf[0] + pf[1]
    al = plr[0] + plr[1]
    inv = 1.0 / jnp.maximum(al[:, CP - 1:CP], 1.0)
    pa = jnp.dot(af * inv, w1[...])
    pred1 = pa + jnp.dot(al[:, :40] * inv, w2[...])
    m = jnp.max(pred1, axis=-1, keepdims=True)
    e = jnp.exp(pred1 - m)
    sm = e / jnp.sum(e, axis=-1, keepdims=True)
    l2 = jnp.where(wr[:, 0:1] > 0.0, sm, lab1[:, :40])
    o_pa[...] = pa
    o_l2[...] = jnp.concatenate(
        [l2, jnp.zeros((l2.shape[0], CP - 41), l2.dtype),
         jnp.ones((l2.shape[0], 1), l2.dtype)], axis=-1)


def _tc_final_body(pl2, pa, w2, o):
    al = pl2[0] + pl2[1]
    inv = 1.0 / jnp.maximum(al[:, CP - 1:CP], 1.0)
    o[...] = pa[...] + jnp.dot(al[:, :40] * inv, w2[...])


def kernel(feat, edge_index, y, mask, val_idx, test_idx, W):
    N, F = feat.shape
    C = W.shape[1]
    E = edge_index.shape[1]
    NPAD = -(-(N + 1) // (8 * NS)) * (8 * NS)
    EPT = E // NT
    KC = 100
    NCH = EPT // KC
    assert EPT % KC == 0 and E % NT == 0

    split = jax.random.uniform(jax.random.key(42), mask.shape) < 0.5
    lrows = jnp.where(split, mask, N).astype(jnp.int32)
    LM = mask.shape[0]
    LPAD = -(-LM // (NS * 96)) * (NS * 96)
    LCH = LPAD // NS // 96
    lrows_p = jnp.full((LPAD,), N, jnp.int32).at[:LM].set(lrows)
    lrows3 = lrows_p.reshape(NS, LCH, 96)

    unl_idx = jnp.concatenate([mask, val_idx, test_idx])
    unl_valid = jnp.concatenate([
        ~split,
        jnp.ones(val_idx.shape, bool),
        jnp.ones(test_idx.shape, bool),
    ])
    unl_rows = jnp.where(unl_valid, unl_idx, N).astype(jnp.int32)
    UM = unl_rows.shape[0]
    UW = 80
    UPAD = -(-UM // (NS * UW)) * (NS * UW)
    UCH = UPAD // NS // UW
    unl_p = jnp.full((UPAD,), N, jnp.int32).at[:UM].set(unl_rows)
    unl3 = unl_p.reshape(NS, UCH, UW)

    src3 = edge_index[0].reshape(NT, NCH, KC).astype(jnp.int32)
    dst3 = edge_index[1].reshape(NT, NCH, KC).astype(jnp.int32)
    y_p = jnp.full((NPAD,), -1, jnp.int32).at[:N].set(y)
    oh40 = (y_p[:, None] == jnp.arange(C)[None, :]).astype(jnp.float32)
    oh_h = jnp.concatenate(
        [oh40, jnp.zeros((NPAD, CP - C - 1), jnp.float32),
         jnp.ones((NPAD, 1), jnp.float32)], axis=-1)
    zc_h = jnp.concatenate(
        [jnp.zeros((NPAD, CP - 1), jnp.float32),
         jnp.ones((NPAD, 1), jnp.float32)], axis=-1)
    zf_h = jnp.zeros((NPAD, F), jnp.float32)
    zl_h = jnp.zeros((NPAD, CP), jnp.float32)
    zd_h = jnp.zeros((NPAD, 16), jnp.float32)
    ones_h = jnp.ones((128, 16), jnp.float32)
    w1 = W[:F]
    w2 = W[F:]

    f32 = jnp.float32
    mesh = plsc.VectorSubcoreMesh(core_axis_name="c", subcore_axis_name="s")
    sc_params = pltpu.CompilerParams(use_tc_tiling_on_sc=False)

    sc_feat = functools.partial(
        pl.kernel,
        functools.partial(_sc_feat_body, NPAD, NCH, KC),
        out_type=[jax.ShapeDtypeStruct((NC, NPAD, F), f32)],
        mesh=mesh,
        compiler_params=sc_params,
        scratch_types=[
            pltpu.VMEM((NCH, KC), jnp.int32),
            pltpu.VMEM((NCH, KC), jnp.int32),
            pltpu.VMEM((KC, F), f32),
            pltpu.VMEM((KC, F), f32),
            pltpu.SemaphoreType.DMA,
            pltpu.SemaphoreType.DMA,
            pltpu.VMEM_SHARED((NPAD, F), f32),
        ],
    )()
    (pf,) = sc_feat(feat, src3, dst3, zf_h)

    sc_label = functools.partial(
        pl.kernel,
        functools.partial(_sc_label_body, NPAD, NCH, KC, LCH, UCH, UW),
        out_type=[
            jax.ShapeDtypeStruct((NC, NPAD, CP), f32),
            jax.ShapeDtypeStruct((NPAD, CP), f32),
            jax.ShapeDtypeStruct((NPAD, 16), f32),
        ],
        mesh=mesh,
        compiler_params=sc_params,
        scratch_types=[
            pltpu.VMEM((NCH, KC), jnp.int32),
            pltpu.VMEM((NCH, KC), jnp.int32),
            pltpu.VMEM((KC, CP), f32),
            pltpu.VMEM((KC, CP), f32),
            pltpu.VMEM((128, 16), f32),
            pltpu.VMEM((96, CP), f32),
            pltpu.VMEM((LCH, 96), jnp.int32),
            pltpu.VMEM((UCH, UW), jnp.int32),
            pltpu.SemaphoreType.DMA,
            pltpu.SemaphoreType.DMA,
            pltpu.VMEM_SHARED((NPAD, CP), f32),
            pltpu.VMEM_SHARED((NPAD, 16), f32),
        ],
    )()
    plr, lab1, wr = sc_label(
        src3, dst3, oh_h, lrows3, unl3, zl_h, zc_h, zd_h, ones_h)

    BR = NPAD // 8
    pa, lab2 = pl.pallas_call(
        _tc_mid_body,
        grid=(NPAD // BR,),
        in_specs=[
            pl.BlockSpec((NC, BR, F), lambda i: (0, i, 0)),
            pl.BlockSpec((NC, BR, CP), lambda i: (0, i, 0)),
            pl.BlockSpec((BR, CP), lambda i: (i, 0)),
            pl.BlockSpec((BR, 16), lambda i: (i, 0)),
            pl.BlockSpec((F, C), lambda i: (0, 0)),
            pl.BlockSpec((C, C), lambda i: (0, 0)),
        ],
        out_specs=[
            pl.BlockSpec((BR, C), lambda i: (i, 0)),
            pl.BlockSpec((BR, CP), lambda i: (i, 0)),
        ],
        out_shape=[
            jax.ShapeDtypeStruct((NPAD, C), f32),
            jax.ShapeDtypeStruct((NPAD, CP), f32),
        ],
    )(pf, plr, lab1, wr, w1, w2)

    sc_lab2 = functools.partial(
        pl.kernel,
        functools.partial(_sc_label2_body, NPAD, NCH, KC),
        out_type=[jax.ShapeDtypeStruct((NC, NPAD, CP), f32)],
        mesh=mesh,
        compiler_params=sc_params,
        scratch_types=[
            pltpu.VMEM((NCH, KC), jnp.int32),
            pltpu.VMEM((NCH, KC), jnp.int32),
            pltpu.VMEM((KC, CP), f32),
            pltpu.VMEM((KC, CP), f32),
            pltpu.SemaphoreType.DMA,
            pltpu.SemaphoreType.DMA,
            pltpu.VMEM_SHARED((NPAD, CP), f32),
        ],
    )()
    (pl2,) = sc_lab2(lab2, src3, dst3, zl_h)

    pred2 = pl.pallas_call(
        _tc_final_body,
        grid=(NPAD // BR,),
        in_specs=[
            pl.BlockSpec((NC, BR, CP), lambda i: (0, i, 0)),
            pl.BlockSpec((BR, C), lambda i: (i, 0)),
            pl.BlockSpec((C, C), lambda i: (0, 0)),
        ],
        out_specs=pl.BlockSpec((BR, C), lambda i: (i, 0)),
        out_shape=jax.ShapeDtypeStruct((NPAD, C), f32),
    )(pl2, pa, w2)

    return pred2[:N]
